# Initial kernel scaffold; baseline (speedup 1.0000x reference)
import jax
import jax.numpy as jnp
from jax.experimental import pallas as pl


def kernel(features, edge_index, W1, b1, W2, b2):
    raise NotImplementedError("write your pallas kernel here")


# speedup vs baseline: 7.2315x; 7.2315x over previous
import functools

import jax
import jax.numpy as jnp
from jax import lax
from jax.experimental import pallas as pl
from jax.experimental.pallas import tpu as pltpu
from jax.experimental.pallas import tpu_sc as plsc

_NC = 2
_NS = 16
_NW = _NC * _NS
_CH = 100


def _make_deg(N, E):
    n_chunks = E // _CH
    per_tile = n_chunks // _NS
    blk = 8
    n_blk = per_tile // blk
    mesh = plsc.VectorSubcoreMesh(core_axis_name="c", subcore_axis_name="s")

    @functools.partial(
        pl.kernel,
        out_type=jax.ShapeDtypeStruct((_NC, N), jnp.float32),
        mesh=mesh,
        scratch_types=[
            pltpu.VMEM((blk, _CH), jnp.int32),
            pltpu.VMEM((_CH,), jnp.float32),
            pltpu.VMEM_SHARED((N,), jnp.float32),
        ],
    )
    def deg_kernel(ei_hbm, zeros_hbm, ones_hbm, out_hbm, idx_blk, ones_v, deg_sh):
        cid = lax.axis_index("c")
        sid = lax.axis_index("s")
        pltpu.sync_copy(ones_hbm, ones_v)

        @pl.when(sid == 0)
        def _():
            pltpu.sync_copy(zeros_hbm, deg_sh)

        plsc.subcore_barrier()

        def outer(bj, carry):
            row0 = pl.multiple_of(sid * per_tile + bj * blk, blk)
            pltpu.sync_copy(ei_hbm.at[cid, pl.ds(row0, blk)], idx_blk)

            def body(j, c2):
                pltpu.sync_copy(ones_v, deg_sh.at[idx_blk.at[j]], add=True)
                return c2

            return lax.fori_loop(0, blk, body, carry)

        lax.fori_loop(0, n_blk, outer, 0)
        plsc.subcore_barrier()

        @pl.when(sid == 0)
        def _():
            pltpu.sync_copy(deg_sh, out_hbm.at[cid])

    return deg_kernel


def _make_prop(N, E, D):
    n_chunks = E // _CH
    per_tile = n_chunks // _NW
    mesh = plsc.VectorSubcoreMesh(core_axis_name="c", subcore_axis_name="s")

    @functools.partial(
        pl.kernel,
        out_type=jax.ShapeDtypeStruct((_NC, N, D), jnp.float32),
        mesh=mesh,
        scratch_types=[
            pltpu.VMEM((per_tile, _CH), jnp.int32),
            pltpu.VMEM((per_tile, _CH), jnp.int32),
            pltpu.VMEM((_CH, D), jnp.float32),
            pltpu.VMEM_SHARED((N, D), jnp.float32),
            pltpu.SemaphoreType.DMA,
        ],
    )
    def prop_kernel(g_hbm, ei_hbm, zeros_hbm, out_hbm,
                    sidx, didx, rows, agg_sh, sem):
        cid = lax.axis_index("c")
        sid = lax.axis_index("s")
        w = cid * _NS + sid
        pltpu.sync_copy(ei_hbm.at[0, w], sidx)
        pltpu.sync_copy(ei_hbm.at[1, w], didx)

        @pl.when(sid == 0)
        def _():
            pltpu.sync_copy(zeros_hbm, agg_sh)

        plsc.subcore_barrier()

        def body(j, carry):
            pltpu.async_copy(g_hbm.at[sidx.at[j]], rows, sem).wait()
            pltpu.sync_copy(rows, agg_sh.at[didx.at[j]], add=True)
            return carry

        lax.fori_loop(0, per_tile, body, 0)
        plsc.subcore_barrier()

        @pl.when(sid == 0)
        def _():
            pltpu.sync_copy(agg_sh, out_hbm.at[cid])

    return prop_kernel


def _tc_b(deg_ref, x_ref, w1_ref, g_ref, ns_ref, nd_ref):
    deg_out = deg_ref[0, :]
    deg_in = deg_ref[1, :]
    ns = jnp.where(deg_out > 0, lax.rsqrt(jnp.maximum(deg_out, 1.0)), 0.0)
    nd = jnp.where(deg_in > 0, lax.rsqrt(jnp.maximum(deg_in, 1.0)), 0.0)
    ns_ref[...] = ns
    nd_ref[...] = nd
    g_ref[...] = jnp.dot(x_ref[...] * ns[:, None], w1_ref[...],
                         preferred_element_type=jnp.float32)


def _tc_c(p_ref, nd_ref, b1_ref, ns_ref, h_ref):
    agg = p_ref[0] + p_ref[1]
    h = agg * nd_ref[...][:, None] + b1_ref[...][None, :]
    h_ref[...] = jnp.maximum(h, 0.0) * ns_ref[...][:, None]


def _tc_d(p_ref, nd_ref, b2_ref, w2_ref, out_ref):
    agg = (p_ref[0] + p_ref[1]) * nd_ref[...][:, None]
    out_ref[...] = jnp.dot(agg, w2_ref[...],
                           preferred_element_type=jnp.float32) + b2_ref[...][None, :]


def kernel(features, edge_index, W1, b1, W2, b2):
    N, D_in = features.shape
    E = edge_index.shape[1]
    D_h = W1.shape[1]
    C = W2.shape[1]

    ei3 = edge_index.reshape(2, E // _CH, _CH)
    ei4 = edge_index.reshape(2, _NW, (E // _CH) // _NW, _CH)
    zeros0 = jnp.zeros((N,), jnp.float32)
    zeros1 = jnp.zeros((N, D_h), jnp.float32)
    ones_ch = jnp.ones((_CH,), jnp.float32)

    deg = _make_deg(N, E)(ei3, zeros0, ones_ch)

    g1, ns, nd = pl.pallas_call(
        _tc_b,
        out_shape=[
            jax.ShapeDtypeStruct((N, D_h), jnp.float32),
            jax.ShapeDtypeStruct((N,), jnp.float32),
            jax.ShapeDtypeStruct((N,), jnp.float32),
        ],
    )(deg, features, W1)

    prop = _make_prop(N, E, D_h)
    p1 = prop(g1, ei4, zeros1)

    h = pl.pallas_call(
        _tc_c,
        out_shape=jax.ShapeDtypeStruct((N, D_h), jnp.float32),
    )(p1, nd, b1, ns)

    p2 = prop(h, ei4, zeros1)

    out = pl.pallas_call(
        _tc_d,
        out_shape=jax.ShapeDtypeStruct((N, C), jnp.float32),
    )(p2, nd, b2, W2)

    return out


# speedup vs baseline: 8.3614x; 1.1562x over previous
import functools

import jax
import jax.numpy as jnp
from jax import lax
from jax.experimental import pallas as pl
from jax.experimental.pallas import tpu as pltpu
from jax.experimental.pallas import tpu_sc as plsc

_NC = 2
_NS = 16
_NW = _NC * _NS
_CH = 100
_CHP = 80


def _make_deg(N, E):
    n_chunks = E // _CH
    per_tile = n_chunks // _NS
    blk = 8
    n_blk = per_tile // blk
    mesh = plsc.VectorSubcoreMesh(core_axis_name="c", subcore_axis_name="s")

    @functools.partial(
        pl.kernel,
        out_type=jax.ShapeDtypeStruct((_NC, N), jnp.float32),
        mesh=mesh,
        scratch_types=[
            pltpu.VMEM((blk, _CH), jnp.int32),
            pltpu.VMEM((_CH,), jnp.float32),
            pltpu.VMEM_SHARED((N,), jnp.float32),
        ],
    )
    def deg_kernel(ei_hbm, zeros_hbm, ones_hbm, out_hbm, idx_blk, ones_v, deg_sh):
        cid = lax.axis_index("c")
        sid = lax.axis_index("s")
        pltpu.sync_copy(ones_hbm, ones_v)

        @pl.when(sid == 0)
        def _():
            pltpu.sync_copy(zeros_hbm, deg_sh)

        plsc.subcore_barrier()

        def outer(bj, carry):
            row0 = pl.multiple_of(sid * per_tile + bj * blk, blk)
            pltpu.sync_copy(ei_hbm.at[cid, pl.ds(row0, blk)], idx_blk)

            def body(j, c2):
                pltpu.sync_copy(ones_v, deg_sh.at[idx_blk.at[j]], add=True)
                return c2

            return lax.fori_loop(0, blk, body, carry)

        lax.fori_loop(0, n_blk, outer, 0)
        plsc.subcore_barrier()

        @pl.when(sid == 0)
        def _():
            pltpu.sync_copy(deg_sh, out_hbm.at[cid])

    return deg_kernel


def _make_prop(N, E, D):
    n_chunks = E // _CHP
    per_tile = n_chunks // _NW
    mesh = plsc.VectorSubcoreMesh(core_axis_name="c", subcore_axis_name="s")

    @functools.partial(
        pl.kernel,
        out_type=jax.ShapeDtypeStruct((_NC, N, D), jnp.float32),
        mesh=mesh,
        scratch_types=[
            pltpu.VMEM((per_tile, _CHP), jnp.int32),
            pltpu.VMEM((_CHP,), jnp.int32),
            pltpu.VMEM((_CHP,), jnp.int32),
            pltpu.VMEM((_CHP,), jnp.int32),
            pltpu.VMEM((_CHP,), jnp.int32),
            pltpu.VMEM((_CHP, D), jnp.float32),
            pltpu.VMEM((_CHP, D), jnp.float32),
            pltpu.VMEM_SHARED((N, D), jnp.float32),
            pltpu.SemaphoreType.DMA,
            pltpu.SemaphoreType.DMA,
        ],
    )
    def prop_kernel(g_hbm, pk_hbm, zeros_hbm, out_hbm,
                    pk, sbuf0, dbuf0, sbuf1, dbuf1,
                    rows0, rows1, agg_sh, sem0, sem1):
        cid = lax.axis_index("c")
        sid = lax.axis_index("s")
        w = cid * _NS + sid
        pltpu.sync_copy(pk_hbm.at[w], pk)

        @pl.when(sid == 0)
        def _():
            pltpu.sync_copy(zeros_hbm, agg_sh)

        def unpack(j, sbuf, dbuf):
            for k in range(_CHP // 16):
                v = pk[j, pl.ds(k * 16, 16)]
                sbuf[pl.ds(k * 16, 16)] = lax.shift_right_logical(v, 16)
                dbuf[pl.ds(k * 16, 16)] = lax.bitwise_and(v, 0xFFFF)

        plsc.subcore_barrier()

        unpack(0, sbuf0, dbuf0)
        pltpu.async_copy(g_hbm.at[sbuf0], rows0, sem0)

        def pair(pj, carry):
            j0 = pj * 2
            j1 = j0 + 1
            j2 = jnp.minimum(j0 + 2, per_tile - 1)
            pltpu.make_async_copy(g_hbm.at[sbuf0], rows0, sem0).wait()
            unpack(j1, sbuf1, dbuf1)
            pltpu.async_copy(g_hbm.at[sbuf1], rows1, sem1)
            pltpu.sync_copy(rows0, agg_sh.at[dbuf0], add=True)
            pltpu.make_async_copy(g_hbm.at[sbuf1], rows1, sem1).wait()
            unpack(j2, sbuf0, dbuf0)
            pltpu.async_copy(g_hbm.at[sbuf0], rows0, sem0)
            pltpu.sync_copy(rows1, agg_sh.at[dbuf1], add=True)
            return carry

        lax.fori_loop(0, per_tile // 2, pair, 0)
        pltpu.make_async_copy(g_hbm.at[sbuf0], rows0, sem0).wait()
        pltpu.sync_copy(rows0, agg_sh.at[dbuf0], add=True)
        plsc.subcore_barrier()

        @pl.when(sid == 0)
        def _():
            pltpu.sync_copy(agg_sh, out_hbm.at[cid])

    return prop_kernel


def _tc_b(deg_ref, x_ref, w1_ref, g_ref, ns_ref, nd_ref):
    deg_out = deg_ref[0, :]
    deg_in = deg_ref[1, :]
    ns = jnp.where(deg_out > 0, lax.rsqrt(jnp.maximum(deg_out, 1.0)), 0.0)
    nd = jnp.where(deg_in > 0, lax.rsqrt(jnp.maximum(deg_in, 1.0)), 0.0)
    ns_ref[...] = ns
    nd_ref[...] = nd
    g_ref[...] = jnp.dot(x_ref[...] * ns[:, None], w1_ref[...],
                         preferred_element_type=jnp.float32)


def _tc_c(p_ref, nd_ref, b1_ref, ns_ref, h_ref):
    agg = p_ref[0] + p_ref[1]
    h = agg * nd_ref[...][:, None] + b1_ref[...][None, :]
    h_ref[...] = jnp.maximum(h, 0.0) * ns_ref[...][:, None]


def _tc_d(p_ref, nd_ref, b2_ref, w2_ref, out_ref):
    agg = (p_ref[0] + p_ref[1]) * nd_ref[...][:, None]
    out_ref[...] = jnp.dot(agg, w2_ref[...],
                           preferred_element_type=jnp.float32) + b2_ref[...][None, :]


def kernel(features, edge_index, W1, b1, W2, b2):
    N, D_in = features.shape
    E = edge_index.shape[1]
    D_h = W1.shape[1]
    C = W2.shape[1]

    ei3 = edge_index.reshape(2, E // _CH, _CH)
    pk4 = ((edge_index[0] << 16) | edge_index[1]).reshape(
        _NW, (E // _CHP) // _NW, _CHP)
    zeros0 = jnp.zeros((N,), jnp.float32)
    zeros1 = jnp.zeros((N, D_h), jnp.float32)
    ones_ch = jnp.ones((_CH,), jnp.float32)

    deg = _make_deg(N, E)(ei3, zeros0, ones_ch)

    g1, ns, nd = pl.pallas_call(
        _tc_b,
        out_shape=[
            jax.ShapeDtypeStruct((N, D_h), jnp.float32),
            jax.ShapeDtypeStruct((N,), jnp.float32),
            jax.ShapeDtypeStruct((N,), jnp.float32),
        ],
    )(deg, features, W1)

    prop = _make_prop(N, E, D_h)
    p1 = prop(g1, pk4, zeros1)

    h = pl.pallas_call(
        _tc_c,
        out_shape=jax.ShapeDtypeStruct((N, D_h), jnp.float32),
    )(p1, nd, b1, ns)

    p2 = prop(h, pk4, zeros1)

    out = pl.pallas_call(
        _tc_d,
        out_shape=jax.ShapeDtypeStruct((N, C), jnp.float32),
    )(p2, nd, b2, W2)

    return out
